# baseline (speedup 1.0000x reference)
import functools

import jax
import jax.numpy as jnp
from jax import lax
from jax.experimental import pallas as pl
from jax.experimental.pallas import tpu as pltpu
from jax.experimental.pallas import tpu_sc as plsc

_B, _S, _D = 4, 2048, 128
_N = _B * _S
_NW = 32
_ROWS = _N // _NW
_NB = _D // 16


def _embed_sum(tok_idx2, seg_f32, token_table, seg_table, pe2d):
    mesh = plsc.VectorSubcoreMesh(core_axis_name="c", subcore_axis_name="s")

    @functools.partial(
        pl.kernel,
        out_type=jax.ShapeDtypeStruct((_N, _D), jnp.float32),
        mesh=mesh,
        scratch_types=[
            pltpu.VMEM((2, _ROWS // 2), jnp.int32),
            pltpu.VMEM((_ROWS,), jnp.float32),
            pltpu.VMEM((_ROWS, _D), jnp.float32),
            pltpu.VMEM((2, _D), jnp.float32),
            pltpu.VMEM((_ROWS, _D), jnp.float32),
            [pltpu.SemaphoreType.DMA] * 2,
            pltpu.SemaphoreType.DMA,
            pltpu.SemaphoreType.DMA,
            pltpu.SemaphoreType.DMA,
        ],
    )
    def k(tok_hbm, seg_hbm, table_hbm, segtab_hbm, pe_hbm, out_hbm,
          tok_v, seg_v, pe_v, st_v, acc_v, gsems, psem, ssem, lsem):
        wid = lax.axis_index("s") * 2 + lax.axis_index("c")
        base = wid * _ROWS
        pbase = lax.rem(base, _S)
        half = _ROWS // 2
        pltpu.sync_copy(tok_hbm.at[wid], tok_v)
        gathers = [
            pltpu.async_copy(table_hbm.at[tok_v.at[j]],
                             acc_v.at[pl.ds(j * half, half)], gsems[j])
            for j in range(2)
        ]
        c_pe = pltpu.async_copy(pe_hbm.at[pl.ds(pbase, _ROWS)], pe_v, psem)
        c_st = pltpu.async_copy(segtab_hbm, st_v, ssem)
        c_lb = pltpu.async_copy(seg_hbm.at[wid], seg_v, lsem)
        c_st.wait()
        c_lb.wait()
        c_pe.wait()
        st0 = [st_v[0, pl.ds(16 * j, 16)] for j in range(_NB)]
        dif = [st_v[1, pl.ds(16 * j, 16)] - st0[j] for j in range(_NB)]
        for g in gathers:
            g.wait()

        def body(g, carry):
            lbl16 = seg_v[pl.ds(16 * g, 16)]
            for t in range(16):
                lbl = jnp.take_along_axis(
                    lbl16, jnp.full((16,), t, jnp.int32), axis=0,
                    mode="promise_in_bounds")
                i = 16 * g + t
                for j in range(_NB):
                    sl = (i, pl.ds(16 * j, 16))
                    acc_v[sl] = acc_v[sl] + pe_v[sl] + (st0[j] + lbl * dif[j])
            return carry

        lax.fori_loop(0, _ROWS // 16, body, 0)
        pltpu.sync_copy(acc_v, out_hbm.at[pl.ds(base, _ROWS)])

    return k(tok_idx2, seg_f32, token_table, seg_table, pe2d)


def kernel(tok_idx, segment_label, token_table, seg_table, pe):
    tok_idx2 = tok_idx.reshape(_NW, 2, _ROWS // 2).astype(jnp.int32)
    seg_f32 = segment_label.reshape(_NW, _ROWS).astype(jnp.float32)
    pe2d = pe.reshape(_S, _D).astype(jnp.float32)
    out = _embed_sum(tok_idx2, seg_f32, token_table, seg_table, pe2d)
    return out.reshape(_B, _S, _D)


# speedup vs baseline: 1.3837x; 1.3837x over previous
import functools

import jax
import jax.numpy as jnp
from jax import lax
from jax.experimental import pallas as pl
from jax.experimental.pallas import tpu as pltpu
from jax.experimental.pallas import tpu_sc as plsc

_B, _S, _D = 4, 2048, 128
_N = _B * _S
_NW = 32
_ROWS = _N // _NW
_NB = _D // 16


def _embed_sum(tok_idx2, seg_f32, token_table, seg_table, pe2d):
    mesh = plsc.VectorSubcoreMesh(core_axis_name="c", subcore_axis_name="s")

    @functools.partial(
        pl.kernel,
        out_type=jax.ShapeDtypeStruct((_N, _D), jnp.float32),
        mesh=mesh,
        scratch_types=[
            pltpu.VMEM((2, _ROWS // 2), jnp.int32),
            pltpu.VMEM((_ROWS,), jnp.float32),
            pltpu.VMEM((_ROWS, _D), jnp.float32),
            pltpu.VMEM((2, _D), jnp.float32),
            pltpu.VMEM((_ROWS, _D), jnp.float32),
            [pltpu.SemaphoreType.DMA] * 2,
            pltpu.SemaphoreType.DMA,
            pltpu.SemaphoreType.DMA,
            pltpu.SemaphoreType.DMA,
        ],
    )
    def k(tok_hbm, seg_hbm, table_hbm, segtab_hbm, pe_hbm, out_hbm,
          tok_v, seg_v, pe_v, st_v, acc_v, gsems, psem, ssem, lsem):
        wid = lax.axis_index("s") * 2 + lax.axis_index("c")
        base = wid * _ROWS
        pbase = lax.rem(base, _S)
        half = _ROWS // 2
        pltpu.sync_copy(tok_hbm.at[wid], tok_v)
        gathers = [
            pltpu.async_copy(table_hbm.at[tok_v.at[j]],
                             acc_v.at[pl.ds(j * half, half)], gsems[j])
            for j in range(2)
        ]
        c_pe = pltpu.async_copy(pe_hbm.at[pl.ds(pbase, _ROWS)], pe_v, psem)
        c_st = pltpu.async_copy(segtab_hbm, st_v, ssem)
        c_lb = pltpu.async_copy(seg_hbm.at[wid], seg_v, lsem)
        c_st.wait()
        c_lb.wait()
        c_pe.wait()
        st0 = [st_v[0, pl.ds(16 * j, 16)] for j in range(_NB)]
        dif = [st_v[1, pl.ds(16 * j, 16)] - st0[j] for j in range(_NB)]
        for g in gathers:
            g.wait()

        @plsc.parallel_loop(0, _ROWS, 1, unroll=4)
        def _merge(i):
            lbl16 = seg_v[pl.ds(16 * lax.div(i, 16), 16)]
            lbl = jnp.take_along_axis(
                lbl16, jnp.full((16,), lax.rem(i, 16), jnp.int32), axis=0,
                mode="promise_in_bounds")
            for j in range(_NB):
                sl = (i, pl.ds(16 * j, 16))
                acc_v[sl] = acc_v[sl] + pe_v[sl] + (st0[j] + lbl * dif[j])
        pltpu.sync_copy(acc_v, out_hbm.at[pl.ds(base, _ROWS)])

    return k(tok_idx2, seg_f32, token_table, seg_table, pe2d)


def kernel(tok_idx, segment_label, token_table, seg_table, pe):
    tok_idx2 = tok_idx.reshape(_NW, 2, _ROWS // 2).astype(jnp.int32)
    seg_f32 = segment_label.reshape(_NW, _ROWS).astype(jnp.float32)
    pe2d = pe.reshape(_S, _D).astype(jnp.float32)
    out = _embed_sum(tok_idx2, seg_f32, token_table, seg_table, pe2d)
    return out.reshape(_B, _S, _D)
